# baseline (speedup 1.0000x reference)
import functools

import jax
import jax.numpy as jnp
from jax import lax
from jax.experimental import pallas as pl
from jax.experimental.pallas import tpu as pltpu
from jax.experimental.pallas import tpu_sc as plsc

HIDDEN_DIM = 64
BETA = 5.0
LANES = 16
NUM_WORKERS = 32
CHUNK = 128


def _sc_boost(emb4, ids3, agency, weights, *, num_steps):
    nw, _, _, d = emb4.shape
    mesh = plsc.VectorSubcoreMesh(core_axis_name="c", subcore_axis_name="s")

    @functools.partial(
        pl.kernel,
        out_type=jax.ShapeDtypeStruct(emb4.shape, jnp.float32),
        mesh=mesh,
        compiler_params=pltpu.CompilerParams(
            needs_layout_passes=False, use_tc_tiling_on_sc=False),
        scratch_types=[
            pltpu.VMEM((num_steps, CHUNK), jnp.int32),
            pltpu.VMEM((CHUNK,), jnp.float32),
            pltpu.VMEM((CHUNK, HIDDEN_DIM), jnp.float32),
            pltpu.VMEM((CHUNK, HIDDEN_DIM), jnp.float32),
            pltpu.SemaphoreType.DMA,
            pltpu.SemaphoreType.DMA,
            pltpu.SemaphoreType.DMA,
        ],
    )
    def k(emb_hbm, ids_hbm, agency_hbm, w_hbm, out_hbm,
          ids_v, w_v, rows_v, emb_v, sem_r, sem_w, sem_e):
        num_cores = jax.lax.axis_size("c")
        wid = lax.axis_index("s") * num_cores + lax.axis_index("c")
        pltpu.sync_copy(ids_hbm.at[wid], ids_v)

        def step(i, _):
            idx = ids_v.at[i]
            cp_r = pltpu.async_copy(agency_hbm.at[idx], rows_v, sem_r)
            cp_w = pltpu.async_copy(w_hbm.at[idx], w_v, sem_w)
            cp_e = pltpu.async_copy(emb_hbm.at[wid, i], emb_v, sem_e)
            cp_r.wait()
            cp_w.wait()
            cp_e.wait()

            def token(t, _):
                tvec = jnp.full((LANES,), t, jnp.int32)
                s = plsc.load_gather(w_v, [tvec]) * BETA
                for j in range(HIDDEN_DIM // LANES):
                    sl = pl.ds(j * LANES, LANES)
                    rows_v[t, sl] = emb_v[t, sl] + s * rows_v[t, sl]
                return 0

            lax.fori_loop(0, CHUNK, token, 0)
            pltpu.sync_copy(rows_v, out_hbm.at[wid, i])
            return 0

        lax.fori_loop(0, num_steps, step, 0)

    return k(emb4, ids3, agency, weights)


def kernel(embeddings, token_ids, agency_matrix, boosting_weights):
    b, l, d = embeddings.shape
    n = b * l
    num_steps = n // (NUM_WORKERS * CHUNK)
    emb4 = embeddings.reshape(NUM_WORKERS, num_steps, CHUNK, d)
    ids3 = token_ids.reshape(NUM_WORKERS, num_steps, CHUNK).astype(jnp.int32)
    out4 = _sc_boost(emb4, ids3, agency_matrix, boosting_weights,
                     num_steps=num_steps)
    return out4.reshape(b, l, d)


# speedup vs baseline: 1.3157x; 1.3157x over previous
import functools

import jax
import jax.numpy as jnp
from jax import lax
from jax.experimental import pallas as pl
from jax.experimental.pallas import tpu as pltpu
from jax.experimental.pallas import tpu_sc as plsc

HIDDEN_DIM = 64
BETA = 5.0
LANES = 16
NUM_WORKERS = 32
CHUNK = 128


def _sc_boost(emb4, ids3, agency, weights, *, num_steps):
    assert num_steps % 2 == 0
    mesh = plsc.VectorSubcoreMesh(core_axis_name="c", subcore_axis_name="s")

    @functools.partial(
        pl.kernel,
        out_type=jax.ShapeDtypeStruct(emb4.shape, jnp.float32),
        mesh=mesh,
        compiler_params=pltpu.CompilerParams(
            needs_layout_passes=False, use_tc_tiling_on_sc=False),
        scratch_types=[
            pltpu.VMEM((num_steps, CHUNK), jnp.int32),
            pltpu.VMEM((CHUNK,), jnp.float32),
            pltpu.VMEM((CHUNK,), jnp.float32),
            pltpu.VMEM((CHUNK, HIDDEN_DIM), jnp.float32),
            pltpu.VMEM((CHUNK, HIDDEN_DIM), jnp.float32),
            pltpu.VMEM((CHUNK, HIDDEN_DIM), jnp.float32),
            pltpu.VMEM((CHUNK, HIDDEN_DIM), jnp.float32),
            pltpu.SemaphoreType.DMA,
            pltpu.SemaphoreType.DMA,
            pltpu.SemaphoreType.DMA,
            pltpu.SemaphoreType.DMA,
        ],
    )
    def k(emb_hbm, ids_hbm, agency_hbm, w_hbm, out_hbm,
          ids_v, w0, w1, rows0, rows1, emb0, emb1,
          sem_in0, sem_in1, sem_out0, sem_out1):
        num_cores = jax.lax.axis_size("c")
        wid = lax.axis_index("s") * num_cores + lax.axis_index("c")
        pltpu.sync_copy(ids_hbm.at[wid], ids_v)

        bufs = ((w0, rows0, emb0, sem_in0, sem_out0),
                (w1, rows1, emb1, sem_in1, sem_out1))

        def issue_in(b, step):
            w_v, rows_v, emb_v, sem_in, _ = bufs[b]
            idx = ids_v.at[step]
            pltpu.async_copy(agency_hbm.at[idx], rows_v, sem_in)
            pltpu.async_copy(w_hbm.at[idx], w_v, sem_in)
            pltpu.async_copy(emb_hbm.at[wid, step], emb_v, sem_in)

        def wait_in(b, step):
            w_v, rows_v, emb_v, sem_in, _ = bufs[b]
            idx = ids_v.at[step]
            pltpu.make_async_copy(agency_hbm.at[idx], rows_v, sem_in).wait()
            pltpu.make_async_copy(w_hbm.at[idx], w_v, sem_in).wait()
            pltpu.make_async_copy(emb_hbm.at[wid, step], emb_v, sem_in).wait()

        def issue_out(b, step):
            _, rows_v, _, _, sem_out = bufs[b]
            pltpu.async_copy(rows_v, out_hbm.at[wid, step], sem_out)

        def wait_out(b, step):
            _, rows_v, _, _, sem_out = bufs[b]
            pltpu.make_async_copy(rows_v, out_hbm.at[wid, step], sem_out).wait()

        def compute(b):
            w_v, rows_v, emb_v, _, _ = bufs[b]
            for g in range(CHUNK // LANES):
                sl = pl.ds(g * LANES, LANES)
                w_v[sl] = w_v[sl] * BETA

            def group(g, _):
                for kk in range(LANES):
                    t = g * LANES + kk
                    tvec = jnp.full((LANES,), t, jnp.int32)
                    s = plsc.load_gather(w_v, [tvec])
                    for j in range(HIDDEN_DIM // LANES):
                        sl = pl.ds(j * LANES, LANES)
                        rows_v[t, sl] = emb_v[t, sl] + s * rows_v[t, sl]
                return 0

            lax.fori_loop(0, CHUNK // LANES, group, 0)

        issue_in(0, 0)

        def pair(ii, _):
            for b in range(2):
                step = 2 * ii + b
                o = 1 - b

                @pl.when(step + 1 < num_steps)
                def _():
                    @pl.when(step >= 1)
                    def _():
                        wait_out(o, step)
                    issue_in(o, step + 1)

                wait_in(b, step)
                compute(b)
                issue_out(b, step)
            return 0

        lax.fori_loop(0, num_steps // 2, pair, 0)
        wait_out(0, num_steps - 2)
        wait_out(1, num_steps - 1)

    return k(emb4, ids3, agency, weights)


def kernel(embeddings, token_ids, agency_matrix, boosting_weights):
    b, l, d = embeddings.shape
    n = b * l
    num_steps = n // (NUM_WORKERS * CHUNK)
    emb4 = embeddings.reshape(NUM_WORKERS, num_steps, CHUNK, d)
    ids3 = token_ids.reshape(NUM_WORKERS, num_steps, CHUNK).astype(jnp.int32)
    out4 = _sc_boost(emb4, ids3, agency_matrix, boosting_weights,
                     num_steps=num_steps)
    return out4.reshape(b, l, d)
